# Initial kernel scaffold; baseline (speedup 1.0000x reference)
import jax
import jax.numpy as jnp
from jax.experimental import pallas as pl


def kernel(features, graph, W1, al1, ar1, b1, W2, al2, ar2, b2, W3, al3, ar3, b3, Wf, bf):
    raise NotImplementedError("write your pallas kernel here")


# speedup vs baseline: 51.0415x; 51.0415x over previous
import functools

import jax
import jax.numpy as jnp
from jax import lax
from jax.experimental import pallas as pl
from jax.experimental.pallas import tpu as pltpu
from jax.experimental.pallas import tpu_sc as plsc

N = 50000
E = 800000
NC = 2
NS = 16
NW = NC * NS
K = 128
SUPROWS = 8
EPT = 25600
EPAD = NW * EPT
EPADK = EPAD // K
ROWS_PER_TILE = EPT // K
SUPS = ROWS_PER_TILE // SUPROWS
NTAB = N + 16
NACC = N + 8
ACC_CHUNK = 200


def _sc_mesh():
    return plsc.VectorSubcoreMesh(
        core_axis_name="c", subcore_axis_name="s", num_cores=NC,
        num_subcores=NS)


_SC_PARAMS = pltpu.CompilerParams(
    needs_layout_passes=False, use_tc_tiling_on_sc=False)


def _sc_ee_kernel(H):
    def body(*refs):
        srcR, dstR = refs[0], refs[1]
        els = refs[2:2 + H]
        ers = refs[2 + H:2 + 2 * H]
        out = refs[2 + 2 * H]
        el_v, er_v, srcc_v, dstc_v, eeb_v = refs[3 + 2 * H:]

        cid = lax.axis_index("c")
        sid = lax.axis_index("s")
        wid = cid * NS + sid

        for h in range(H):
            pltpu.sync_copy(els[h], el_v)
            pltpu.sync_copy(ers[h], er_v)

            def sup_body(s, _):
                row0 = wid * ROWS_PER_TILE + s * SUPROWS
                pltpu.sync_copy(srcR.at[pl.ds(row0, SUPROWS)], srcc_v)
                pltpu.sync_copy(dstR.at[pl.ds(row0, SUPROWS)], dstc_v)
                for j in range(SUPROWS):
                    for g in range(8):
                        s16 = srcc_v[j, pl.ds(g * 16, 16)]
                        d16 = dstc_v[j, pl.ds(g * 16, 16)]
                        el16 = plsc.load_gather(el_v, [s16])
                        er16 = plsc.load_gather(er_v, [d16])
                        t = el16 + er16
                        e = jnp.maximum(t, 0.2 * t)
                        eeb_v[j, pl.ds(g * 16, 16)] = jnp.exp(e)
                pltpu.sync_copy(eeb_v, out.at[h, pl.ds(row0, SUPROWS)])
                return ()
            lax.fori_loop(0, SUPS, sup_body, ())

    scratch = [
        pltpu.VMEM((NTAB,), jnp.float32),
        pltpu.VMEM((NTAB,), jnp.float32),
        pltpu.VMEM((SUPROWS, K), jnp.int32),
        pltpu.VMEM((SUPROWS, K), jnp.int32),
        pltpu.VMEM((SUPROWS, K), jnp.float32),
    ]
    return pl.kernel(
        body,
        out_type=jax.ShapeDtypeStruct((H, EPADK, K), jnp.float32),
        mesh=_sc_mesh(),
        scratch_types=scratch,
        compiler_params=_SC_PARAMS,
    )


def _sc_agg_kernel(H):
    def body(*refs):
        srcR, dstR, eeP = refs[0], refs[1], refs[2]
        featPs = refs[3:3 + H]
        out = refs[3 + H]
        (srcc_v, dstc_v, eeb_v, rows0, rows1, zbuf, acc,
         gsem0, gsem1) = refs[4 + H:]
        rows = (rows0, rows1)
        gsems = (gsem0, gsem1)

        cid = lax.axis_index("c")
        sid = lax.axis_index("s")
        wid = cid * NS + sid

        def zb_body(kk, _):
            zbuf[kk, pl.ds(0, 16)] = jnp.zeros((16,), jnp.float32)
            zbuf[kk, pl.ds(16, 16)] = jnp.zeros((16,), jnp.float32)
            return ()
        lax.fori_loop(0, ACC_CHUNK, zb_body, (), unroll=4)

        nch = jnp.where(sid < 15, 16, 10)
        r0 = sid * 3200

        for h in range(H):
            def zero_body(z, _):
                pltpu.sync_copy(
                    zbuf, acc.at[pl.ds(r0 + z * ACC_CHUNK, ACC_CHUNK)])
                return ()
            lax.fori_loop(0, nch, zero_body, ())

            @pl.when(sid == 0)
            def _():
                pltpu.sync_copy(zbuf.at[pl.ds(0, 8)], acc.at[pl.ds(N, 8)])
            plsc.subcore_barrier()

            def sup_body(s, _):
                row0 = wid * ROWS_PER_TILE + s * SUPROWS
                pltpu.sync_copy(srcR.at[pl.ds(row0, SUPROWS)], srcc_v)
                pltpu.sync_copy(dstR.at[pl.ds(row0, SUPROWS)], dstc_v)
                pltpu.sync_copy(eeP.at[h, pl.ds(row0, SUPROWS)], eeb_v)

                descs = [None, None]
                descs[0] = pltpu.async_copy(
                    featPs[h].at[srcc_v.at[0]], rows[0], gsems[0])
                for j in range(SUPROWS):
                    p = j % 2
                    if j + 1 < SUPROWS:
                        descs[1 - p] = pltpu.async_copy(
                            featPs[h].at[srcc_v.at[j + 1]],
                            rows[1 - p], gsems[1 - p])
                    descs[p].wait()

                    def scale_body(kk, _):
                        av = plsc.load_gather(
                            eeb_v, [jnp.full((16,), j, jnp.int32),
                                    jnp.full((16,), kk, jnp.int32)])
                        rows[p][kk, pl.ds(0, 16)] = \
                            rows[p][kk, pl.ds(0, 16)] * av
                        rows[p][kk, pl.ds(16, 16)] = \
                            rows[p][kk, pl.ds(16, 16)] * av
                        return ()
                    lax.fori_loop(0, K, scale_body, (), unroll=4)

                    pltpu.sync_copy(rows[p], acc.at[dstc_v.at[j]],
                                    add=True)
                return ()
            lax.fori_loop(0, SUPS, sup_body, ())
            plsc.subcore_barrier()

            def copy_body(z, _):
                pltpu.sync_copy(
                    acc.at[pl.ds(r0 + z * ACC_CHUNK, ACC_CHUNK)],
                    out.at[cid, h, pl.ds(r0 + z * ACC_CHUNK, ACC_CHUNK)])
                return ()
            lax.fori_loop(0, nch, copy_body, ())
            plsc.subcore_barrier()

    scratch = [
        pltpu.VMEM((SUPROWS, K), jnp.int32),
        pltpu.VMEM((SUPROWS, K), jnp.int32),
        pltpu.VMEM((SUPROWS, K), jnp.float32),
        pltpu.VMEM((K, 32), jnp.float32),
        pltpu.VMEM((K, 32), jnp.float32),
        pltpu.VMEM((ACC_CHUNK, 32), jnp.float32),
        pltpu.VMEM_SHARED((NACC, 32), jnp.float32),
        pltpu.SemaphoreType.DMA,
        pltpu.SemaphoreType.DMA,
    ]
    return pl.kernel(
        body,
        out_type=jax.ShapeDtypeStruct((NC, H, N, 32), jnp.float32),
        mesh=_sc_mesh(),
        scratch_types=scratch,
        compiler_params=_SC_PARAMS,
    )


def _tc_prep1(x, W, AL, AR):
    B = 2000
    Fin, Fout = W.shape
    Hh = AL.shape[1]

    def body(x_ref, w_ref, al_ref, ar_ref, feat_ref, el_ref, er_ref):
        feat = jnp.dot(x_ref[...], w_ref[...],
                       preferred_element_type=jnp.float32)
        feat_ref[...] = feat
        el_ref[...] = jnp.dot(feat, al_ref[...],
                              preferred_element_type=jnp.float32)
        er_ref[...] = jnp.dot(feat, ar_ref[...],
                              preferred_element_type=jnp.float32)

    return pl.pallas_call(
        body,
        grid=(N // B,),
        in_specs=[
            pl.BlockSpec((B, Fin), lambda i: (i, 0)),
            pl.BlockSpec((Fin, Fout), lambda i: (0, 0)),
            pl.BlockSpec((Fout, Hh), lambda i: (0, 0)),
            pl.BlockSpec((Fout, Hh), lambda i: (0, 0)),
        ],
        out_specs=[
            pl.BlockSpec((B, Fout), lambda i: (i, 0)),
            pl.BlockSpec((B, Hh), lambda i: (i, 0)),
            pl.BlockSpec((B, Hh), lambda i: (i, 0)),
        ],
        out_shape=[
            jax.ShapeDtypeStruct((N, Fout), jnp.float32),
            jax.ShapeDtypeStruct((N, Hh), jnp.float32),
            jax.ShapeDtypeStruct((N, Hh), jnp.float32),
        ],
    )(x, W, AL, AR)


def _tc_prep_next(p0, p1, b, W, AL, AR):
    B = 2000
    Hp = p0.shape[0]
    Fin, Fout = W.shape
    Hh = AL.shape[1]

    def body(p0_ref, p1_ref, b_ref, w_ref, al_ref, ar_ref,
             feat_ref, el_ref, er_ref):
        num = p0_ref[...] + p1_ref[...]
        den = num[:, :, 25:26] + 1e-16
        nrm = num / den
        hcat = jnp.concatenate([nrm[h, :, :25] for h in range(Hp)],
                               axis=-1) + b_ref[...]
        feat = jnp.dot(hcat, w_ref[...],
                       preferred_element_type=jnp.float32)
        feat_ref[...] = feat
        el_ref[...] = jnp.dot(feat, al_ref[...],
                              preferred_element_type=jnp.float32)
        er_ref[...] = jnp.dot(feat, ar_ref[...],
                              preferred_element_type=jnp.float32)

    return pl.pallas_call(
        body,
        grid=(N // B,),
        in_specs=[
            pl.BlockSpec((Hp, B, 32), lambda i: (0, i, 0)),
            pl.BlockSpec((Hp, B, 32), lambda i: (0, i, 0)),
            pl.BlockSpec((1, Fin), lambda i: (0, 0)),
            pl.BlockSpec((Fin, Fout), lambda i: (0, 0)),
            pl.BlockSpec((Fout, Hh), lambda i: (0, 0)),
            pl.BlockSpec((Fout, Hh), lambda i: (0, 0)),
        ],
        out_specs=[
            pl.BlockSpec((B, Fout), lambda i: (i, 0)),
            pl.BlockSpec((B, Hh), lambda i: (i, 0)),
            pl.BlockSpec((B, Hh), lambda i: (i, 0)),
        ],
        out_shape=[
            jax.ShapeDtypeStruct((N, Fout), jnp.float32),
            jax.ShapeDtypeStruct((N, Hh), jnp.float32),
            jax.ShapeDtypeStruct((N, Hh), jnp.float32),
        ],
    )(p0, p1, b, W, AL, AR)


def _tc_final(p0, p1, b, Wf, bf):
    B = 2000

    def body(p0_ref, p1_ref, b_ref, wf_ref, bf_ref, out_ref):
        num = p0_ref[...] + p1_ref[...]
        den = num[:, :, 25:26] + 1e-16
        h3 = (num / den)[0, :, :25] + b_ref[...]
        z = jnp.dot(h3, wf_ref[...],
                    preferred_element_type=jnp.float32) + bf_ref[...]
        out_ref[...] = 1.0 / (1.0 + jnp.exp(-z))

    return pl.pallas_call(
        body,
        grid=(N // B,),
        in_specs=[
            pl.BlockSpec((1, B, 32), lambda i: (0, i, 0)),
            pl.BlockSpec((1, B, 32), lambda i: (0, i, 0)),
            pl.BlockSpec((1, 25), lambda i: (0, 0)),
            pl.BlockSpec((25, 1), lambda i: (0, 0)),
            pl.BlockSpec((1, 1), lambda i: (0, 0)),
        ],
        out_specs=pl.BlockSpec((B, 1), lambda i: (i, 0)),
        out_shape=jax.ShapeDtypeStruct((N, 1), jnp.float32),
    )(p0, p1, b, Wf, bf)


def _attn_mats(al, ar):
    H, D = al.shape
    AL = jnp.zeros((H * D, H), jnp.float32)
    AR = jnp.zeros((H * D, H), jnp.float32)
    for h in range(H):
        AL = AL.at[h * D:(h + 1) * D, h].set(al[h])
        AR = AR.at[h * D:(h + 1) * D, h].set(ar[h])
    return AL, AR


def _pack_tables(feat, el, er, H):
    ones = jnp.ones((N, 1), jnp.float32)
    zeros = jnp.zeros((N, 6), jnp.float32)
    featPs, els, ers = [], [], []
    for h in range(H):
        featPs.append(jnp.concatenate(
            [feat[:, h * 25:(h + 1) * 25], ones, zeros], axis=1))
        els.append(jnp.pad(el[:, h], (0, NTAB - N)))
        ers.append(jnp.pad(er[:, h], (0, NTAB - N)))
    return featPs, els, ers


def kernel(features, graph, W1, al1, ar1, b1, W2, al2, ar2, b2,
           W3, al3, ar3, b3, Wf, bf):
    src_p = jnp.pad(graph[0], (0, EPAD - E)).reshape(EPAD // K, K)
    dst_p = jnp.pad(graph[1], (0, EPAD - E),
                    constant_values=N).reshape(EPAD // K, K)

    AL1, AR1 = _attn_mats(al1, ar1)
    AL2, AR2 = _attn_mats(al2, ar2)
    AL3, AR3 = _attn_mats(al3, ar3)

    feat, el, er = _tc_prep1(features, W1, AL1, AR1)
    featPs, els, ers = _pack_tables(feat, el, er, 4)
    ee = _sc_ee_kernel(4)(src_p, dst_p, *els, *ers)
    p = _sc_agg_kernel(4)(src_p, dst_p, ee, *featPs)

    feat, el, er = _tc_prep_next(p[0], p[1], b1.reshape(1, -1), W2, AL2, AR2)
    featPs, els, ers = _pack_tables(feat, el, er, 2)
    ee = _sc_ee_kernel(2)(src_p, dst_p, *els, *ers)
    p = _sc_agg_kernel(2)(src_p, dst_p, ee, *featPs)

    feat, el, er = _tc_prep_next(p[0], p[1], b2.reshape(1, -1), W3, AL3, AR3)
    featPs, els, ers = _pack_tables(feat, el, er, 1)
    ee = _sc_ee_kernel(1)(src_p, dst_p, *els, *ers)
    p = _sc_agg_kernel(1)(src_p, dst_p, ee, *featPs)

    return _tc_final(p[0], p[1], b3.reshape(1, -1), Wf, bf.reshape(1, 1))


# speedup vs baseline: 66.8935x; 1.3106x over previous
import functools

import jax
import jax.numpy as jnp
from jax import lax
from jax.experimental import pallas as pl
from jax.experimental.pallas import tpu as pltpu
from jax.experimental.pallas import tpu_sc as plsc

N = 50000
E = 800000
NC = 2
NS = 16
NW = NC * NS
K = 128
SUPROWS = 8
EPT = 25600
EPAD = NW * EPT
EPADK = EPAD // K
ROWS_PER_TILE = EPT // K
SUPS = ROWS_PER_TILE // SUPROWS
NJUNK = 512
NTAB = N + NJUNK
NACC = N + NJUNK
ACC_CHUNK = 200


def _sc_mesh():
    return plsc.VectorSubcoreMesh(
        core_axis_name="c", subcore_axis_name="s", num_cores=NC,
        num_subcores=NS)


_SC_PARAMS = pltpu.CompilerParams(
    needs_layout_passes=False, use_tc_tiling_on_sc=False)


def _sc_ee_kernel(H):
    def body(*refs):
        srcR, dstR, elT, erT = refs[:4]
        out = refs[4]
        el_v, er_v, srcc_v, dstc_v, eeb_v = refs[5:]

        cid = lax.axis_index("c")
        sid = lax.axis_index("s")
        wid = cid * NS + sid

        def zt_body(kk, _):
            el_v[pl.ds(N + kk * 16, 16)] = jnp.zeros((16,), jnp.float32)
            er_v[pl.ds(N + kk * 16, 16)] = jnp.zeros((16,), jnp.float32)
            return ()
        lax.fori_loop(0, NJUNK // 16, zt_body, (), unroll=4)

        for h in range(H):
            pltpu.sync_copy(elT.at[h], el_v.at[pl.ds(0, N)])
            pltpu.sync_copy(erT.at[h], er_v.at[pl.ds(0, N)])

            def sup_body(s, _):
                row0 = wid * ROWS_PER_TILE + s * SUPROWS
                pltpu.sync_copy(srcR.at[pl.ds(row0, SUPROWS)], srcc_v)
                pltpu.sync_copy(dstR.at[pl.ds(row0, SUPROWS)], dstc_v)
                for j in range(SUPROWS):
                    for g in range(8):
                        s16 = srcc_v[j, pl.ds(g * 16, 16)]
                        d16 = dstc_v[j, pl.ds(g * 16, 16)]
                        el16 = plsc.load_gather(el_v, [s16])
                        er16 = plsc.load_gather(er_v, [d16])
                        t = el16 + er16
                        e = jnp.maximum(t, 0.2 * t)
                        eeb_v[j, pl.ds(g * 16, 16)] = jnp.exp(e)
                pltpu.sync_copy(eeb_v, out.at[h, pl.ds(row0, SUPROWS)])
                return ()
            lax.fori_loop(0, SUPS, sup_body, ())

    scratch = [
        pltpu.VMEM((NTAB,), jnp.float32),
        pltpu.VMEM((NTAB,), jnp.float32),
        pltpu.VMEM((SUPROWS, K), jnp.int32),
        pltpu.VMEM((SUPROWS, K), jnp.int32),
        pltpu.VMEM((SUPROWS, K), jnp.float32),
    ]
    return pl.kernel(
        body,
        out_type=jax.ShapeDtypeStruct((H, EPADK, K), jnp.float32),
        mesh=_sc_mesh(),
        scratch_types=scratch,
        compiler_params=_SC_PARAMS,
    )


def _sc_agg_kernel(H):
    def body(*refs):
        srcR, dstR, eeP = refs[0], refs[1], refs[2]
        featPs = refs[3:3 + H]
        out = refs[3 + H]
        (srcc_v, dstc_v, eeb_v, rows0, rows1, zbuf, acc,
         gsem0, gsem1, ssem0, ssem1) = refs[4 + H:]
        rows = (rows0, rows1)
        gsems = (gsem0, gsem1)
        ssems = (ssem0, ssem1)

        cid = lax.axis_index("c")
        sid = lax.axis_index("s")
        wid = cid * NS + sid

        def zb_body(kk, _):
            zbuf[kk, pl.ds(0, 16)] = jnp.zeros((16,), jnp.float32)
            zbuf[kk, pl.ds(16, 16)] = jnp.zeros((16,), jnp.float32)
            return ()
        lax.fori_loop(0, ACC_CHUNK, zb_body, (), unroll=4)

        nch = jnp.where(sid < 15, 16, 10)
        r0 = sid * 3200

        for h in range(H):
            def zero_body(z, _):
                pltpu.sync_copy(
                    zbuf, acc.at[pl.ds(r0 + z * ACC_CHUNK, ACC_CHUNK)])
                return ()
            lax.fori_loop(0, nch, zero_body, ())
            plsc.subcore_barrier()

            def sup_body(s, _):
                row0 = wid * ROWS_PER_TILE + s * SUPROWS
                pltpu.sync_copy(srcR.at[pl.ds(row0, SUPROWS)], srcc_v)
                pltpu.sync_copy(dstR.at[pl.ds(row0, SUPROWS)], dstc_v)
                pltpu.sync_copy(eeP.at[h, pl.ds(row0, SUPROWS)], eeb_v)

                gd = [None, None]
                sd = [None, None]
                gd[0] = pltpu.async_copy(
                    featPs[h].at[srcc_v.at[0]], rows[0], gsems[0])
                for j in range(SUPROWS):
                    p = j % 2
                    if j + 1 < SUPROWS:
                        if sd[1 - p] is not None:
                            sd[1 - p].wait()
                        gd[1 - p] = pltpu.async_copy(
                            featPs[h].at[srcc_v.at[j + 1]],
                            rows[1 - p], gsems[1 - p])
                    gd[p].wait()

                    def scale_body(kk, _):
                        av = plsc.load_gather(
                            eeb_v, [jnp.full((16,), j, jnp.int32),
                                    jnp.full((16,), kk, jnp.int32)])
                        rows[p][kk, pl.ds(0, 16)] = \
                            rows[p][kk, pl.ds(0, 16)] * av
                        rows[p][kk, pl.ds(16, 16)] = \
                            rows[p][kk, pl.ds(16, 16)] * av
                        return ()
                    lax.fori_loop(0, K, scale_body, (), unroll=8)

                    sd[p] = pltpu.async_copy(
                        rows[p], acc.at[dstc_v.at[j]], ssems[p], add=True)
                sd[0].wait()
                sd[1].wait()
                return ()
            lax.fori_loop(0, SUPS, sup_body, ())
            plsc.subcore_barrier()

            def copy_body(z, _):
                pltpu.sync_copy(
                    acc.at[pl.ds(r0 + z * ACC_CHUNK, ACC_CHUNK)],
                    out.at[cid, h, pl.ds(r0 + z * ACC_CHUNK, ACC_CHUNK)])
                return ()
            lax.fori_loop(0, nch, copy_body, ())
            plsc.subcore_barrier()

    scratch = [
        pltpu.VMEM((SUPROWS, K), jnp.int32),
        pltpu.VMEM((SUPROWS, K), jnp.int32),
        pltpu.VMEM((SUPROWS, K), jnp.float32),
        pltpu.VMEM((K, 32), jnp.float32),
        pltpu.VMEM((K, 32), jnp.float32),
        pltpu.VMEM((ACC_CHUNK, 32), jnp.float32),
        pltpu.VMEM_SHARED((NACC, 32), jnp.float32),
        pltpu.SemaphoreType.DMA,
        pltpu.SemaphoreType.DMA,
        pltpu.SemaphoreType.DMA,
        pltpu.SemaphoreType.DMA,
    ]
    return pl.kernel(
        body,
        out_type=jax.ShapeDtypeStruct((NC, H, N, 32), jnp.float32),
        mesh=_sc_mesh(),
        scratch_types=scratch,
        compiler_params=_SC_PARAMS,
    )


def _emit_tables(feat, al_ref, ar_ref, sel_ref, c_ref, H,
                 featP_refs, elT_ref, erT_ref):
    elT_ref[...] = jnp.dot(feat, al_ref[...],
                           preferred_element_type=jnp.float32)
    erT_ref[...] = jnp.dot(feat, ar_ref[...],
                           preferred_element_type=jnp.float32)
    for h in range(H):
        featP_refs[h][...] = jnp.dot(
            feat, sel_ref[h], preferred_element_type=jnp.float32) + c_ref[...]


def _tc_prep1(x, W, AL, AR, S, c):
    B = 2000
    Fin, Fout = W.shape
    Hh = AL.shape[1]

    def body(x_ref, w_ref, al_ref, ar_ref, sel_ref, c_ref, *out_refs):
        feat = jnp.dot(x_ref[...], w_ref[...],
                       preferred_element_type=jnp.float32)
        _emit_tables(feat, al_ref, ar_ref, sel_ref, c_ref, Hh,
                     out_refs[:Hh], out_refs[Hh], out_refs[Hh + 1])

    return pl.pallas_call(
        body,
        grid=(N // B,),
        in_specs=[
            pl.BlockSpec((B, Fin), lambda i: (i, 0)),
            pl.BlockSpec((Fin, Fout), lambda i: (0, 0)),
            pl.BlockSpec((Fout, Hh), lambda i: (0, 0)),
            pl.BlockSpec((Fout, Hh), lambda i: (0, 0)),
            pl.BlockSpec((Hh, Fout, 32), lambda i: (0, 0, 0)),
            pl.BlockSpec((1, 32), lambda i: (0, 0)),
        ],
        out_specs=(
            [pl.BlockSpec((B, 32), lambda i: (i, 0)) for _ in range(Hh)]
            + [pl.BlockSpec((B, Hh), lambda i: (i, 0)),
               pl.BlockSpec((B, Hh), lambda i: (i, 0))]
        ),
        out_shape=(
            [jax.ShapeDtypeStruct((N, 32), jnp.float32) for _ in range(Hh)]
            + [jax.ShapeDtypeStruct((N, Hh), jnp.float32),
               jax.ShapeDtypeStruct((N, Hh), jnp.float32)]
        ),
    )(x, W, AL, AR, S, c)


def _tc_prep_next(p0, p1, b, W, AL, AR, S, c):
    B = 2000
    Hp = p0.shape[0]
    Fin, Fout = W.shape
    Hh = AL.shape[1]

    def body(p0_ref, p1_ref, b_ref, w_ref, al_ref, ar_ref, sel_ref, c_ref,
             *out_refs):
        num = p0_ref[...] + p1_ref[...]
        den = num[:, :, 25:26] + 1e-16
        nrm = num / den
        hcat = jnp.concatenate([nrm[h, :, :25] for h in range(Hp)],
                               axis=-1) + b_ref[...]
        feat = jnp.dot(hcat, w_ref[...],
                       preferred_element_type=jnp.float32)
        _emit_tables(feat, al_ref, ar_ref, sel_ref, c_ref, Hh,
                     out_refs[:Hh], out_refs[Hh], out_refs[Hh + 1])

    return pl.pallas_call(
        body,
        grid=(N // B,),
        in_specs=[
            pl.BlockSpec((Hp, B, 32), lambda i: (0, i, 0)),
            pl.BlockSpec((Hp, B, 32), lambda i: (0, i, 0)),
            pl.BlockSpec((1, Fin), lambda i: (0, 0)),
            pl.BlockSpec((Fin, Fout), lambda i: (0, 0)),
            pl.BlockSpec((Fout, Hh), lambda i: (0, 0)),
            pl.BlockSpec((Fout, Hh), lambda i: (0, 0)),
            pl.BlockSpec((Hh, Fout, 32), lambda i: (0, 0, 0)),
            pl.BlockSpec((1, 32), lambda i: (0, 0)),
        ],
        out_specs=(
            [pl.BlockSpec((B, 32), lambda i: (i, 0)) for _ in range(Hh)]
            + [pl.BlockSpec((B, Hh), lambda i: (i, 0)),
               pl.BlockSpec((B, Hh), lambda i: (i, 0))]
        ),
        out_shape=(
            [jax.ShapeDtypeStruct((N, 32), jnp.float32) for _ in range(Hh)]
            + [jax.ShapeDtypeStruct((N, Hh), jnp.float32),
               jax.ShapeDtypeStruct((N, Hh), jnp.float32)]
        ),
    )(p0, p1, b, W, AL, AR, S, c)


def _tc_final(p0, p1, b, Wf, bf):
    B = 2000

    def body(p0_ref, p1_ref, b_ref, wf_ref, bf_ref, out_ref):
        num = p0_ref[...] + p1_ref[...]
        den = num[:, :, 25:26] + 1e-16
        h3 = (num / den)[0, :, :25] + b_ref[...]
        z = jnp.dot(h3, wf_ref[...],
                    preferred_element_type=jnp.float32) + bf_ref[...]
        out_ref[...] = 1.0 / (1.0 + jnp.exp(-z))

    return pl.pallas_call(
        body,
        grid=(N // B,),
        in_specs=[
            pl.BlockSpec((1, B, 32), lambda i: (0, i, 0)),
            pl.BlockSpec((1, B, 32), lambda i: (0, i, 0)),
            pl.BlockSpec((1, 25), lambda i: (0, 0)),
            pl.BlockSpec((25, 1), lambda i: (0, 0)),
            pl.BlockSpec((1, 1), lambda i: (0, 0)),
        ],
        out_specs=pl.BlockSpec((B, 1), lambda i: (i, 0)),
        out_shape=jax.ShapeDtypeStruct((N, 1), jnp.float32),
    )(p0, p1, b, Wf, bf)


def _attn_mats(al, ar):
    H, D = al.shape
    AL = jnp.zeros((H * D, H), jnp.float32)
    AR = jnp.zeros((H * D, H), jnp.float32)
    S = jnp.zeros((H, H * D, 32), jnp.float32)
    for h in range(H):
        AL = AL.at[h * D:(h + 1) * D, h].set(al[h])
        AR = AR.at[h * D:(h + 1) * D, h].set(ar[h])
        S = S.at[h, h * D:(h + 1) * D, :D].set(jnp.eye(D, dtype=jnp.float32))
    c = jnp.zeros((1, 32), jnp.float32).at[0, 25].set(1.0)
    return AL, AR, S, c


def kernel(features, graph, W1, al1, ar1, b1, W2, al2, ar2, b2,
           W3, al3, ar3, b3, Wf, bf):
    pad = EPAD - E
    src_p = jnp.pad(graph[0], (0, pad),
                    mode="wrap").reshape(EPAD // K, K)
    dst_p = (jnp.pad(graph[1], (0, pad), constant_values=0)
             .at[E:].set(N + (jnp.arange(pad, dtype=jnp.int32) % NJUNK))
             .reshape(EPAD // K, K))

    AL1, AR1, S1, c1 = _attn_mats(al1, ar1)
    AL2, AR2, S2, c2 = _attn_mats(al2, ar2)
    AL3, AR3, S3, c3 = _attn_mats(al3, ar3)

    *featPs, el, er = _tc_prep1(features, W1, AL1, AR1, S1, c1)
    ee = _sc_ee_kernel(4)(src_p, dst_p, el.T, er.T)
    p = _sc_agg_kernel(4)(src_p, dst_p, ee, *featPs)

    *featPs, el, er = _tc_prep_next(
        p[0], p[1], b1.reshape(1, -1), W2, AL2, AR2, S2, c2)
    ee = _sc_ee_kernel(2)(src_p, dst_p, el.T, er.T)
    p = _sc_agg_kernel(2)(src_p, dst_p, ee, *featPs)

    *featPs, el, er = _tc_prep_next(
        p[0], p[1], b2.reshape(1, -1), W3, AL3, AR3, S3, c3)
    ee = _sc_ee_kernel(1)(src_p, dst_p, el.T, er.T)
    p = _sc_agg_kernel(1)(src_p, dst_p, ee, *featPs)

    return _tc_final(p[0], p[1], b3.reshape(1, -1), Wf, bf.reshape(1, 1))


# speedup vs baseline: 71.7663x; 1.0728x over previous
import functools

import jax
import jax.numpy as jnp
from jax import lax
from jax.experimental import pallas as pl
from jax.experimental.pallas import tpu as pltpu
from jax.experimental.pallas import tpu_sc as plsc

N = 50000
E = 800000
NC = 2
NS = 16
NW = NC * NS
K = 128
SUPROWS = 40
EPT = 25600
EPAD = NW * EPT
EPADK = EPAD // K
ROWS_PER_TILE = EPT // K
SUPS = ROWS_PER_TILE // SUPROWS
NJUNK = 512
NTAB = N + NJUNK
NACC = N + NJUNK


def _sc_mesh():
    return plsc.VectorSubcoreMesh(
        core_axis_name="c", subcore_axis_name="s", num_cores=NC,
        num_subcores=NS)


_SC_PARAMS = pltpu.CompilerParams(
    needs_layout_passes=False, use_tc_tiling_on_sc=False)


def _sc_ee_kernel(H):
    def body(*refs):
        srcR, dstR, elT, erT = refs[:4]
        out = refs[4]
        el_v, er_v, srcc_v, dstc_v, eeb_v = refs[5:]

        cid = lax.axis_index("c")
        sid = lax.axis_index("s")
        wid = cid * NS + sid

        def zt_body(kk, _):
            el_v[pl.ds(N + kk * 16, 16)] = jnp.zeros((16,), jnp.float32)
            er_v[pl.ds(N + kk * 16, 16)] = jnp.zeros((16,), jnp.float32)
            return ()
        lax.fori_loop(0, NJUNK // 16, zt_body, (), unroll=4)

        for h in range(H):
            pltpu.sync_copy(elT.at[h], el_v.at[pl.ds(0, N)])
            pltpu.sync_copy(erT.at[h], er_v.at[pl.ds(0, N)])

            def sup_body(s, _):
                row0 = wid * ROWS_PER_TILE + s * SUPROWS
                pltpu.sync_copy(srcR.at[pl.ds(row0, SUPROWS)], srcc_v)
                pltpu.sync_copy(dstR.at[pl.ds(row0, SUPROWS)], dstc_v)

                def chunk_body(j, _):
                    for g in range(8):
                        s16 = srcc_v[j, pl.ds(g * 16, 16)]
                        d16 = dstc_v[j, pl.ds(g * 16, 16)]
                        el16 = plsc.load_gather(el_v, [s16])
                        er16 = plsc.load_gather(er_v, [d16])
                        t = el16 + er16
                        e = jnp.maximum(t, 0.2 * t)
                        eeb_v[j, pl.ds(g * 16, 16)] = jnp.exp(e)
                    return ()
                lax.fori_loop(0, SUPROWS, chunk_body, ())
                pltpu.sync_copy(eeb_v, out.at[h, pl.ds(row0, SUPROWS)])
                return ()
            lax.fori_loop(0, SUPS, sup_body, ())

    scratch = [
        pltpu.VMEM((NTAB,), jnp.float32),
        pltpu.VMEM((NTAB,), jnp.float32),
        pltpu.VMEM((SUPROWS, K), jnp.int32),
        pltpu.VMEM((SUPROWS, K), jnp.int32),
        pltpu.VMEM((SUPROWS, K), jnp.float32),
    ]
    return pl.kernel(
        body,
        out_type=jax.ShapeDtypeStruct((H, EPADK, K), jnp.float32),
        mesh=_sc_mesh(),
        scratch_types=scratch,
        compiler_params=_SC_PARAMS,
    )


def _sc_agg_kernel(H):
    def body(*refs):
        srcR, dstR, eeP = refs[0], refs[1], refs[2]
        featPs = refs[3:3 + H]
        out = refs[3 + H]
        (srcc_v, dstc_v, eeb_v, rows0, rows1, acc,
         gsem0, gsem1, ssem0, ssem1) = refs[4 + H:]
        rows = (rows0, rows1)
        gsems = (gsem0, gsem1)
        ssems = (ssem0, ssem1)

        cid = lax.axis_index("c")
        sid = lax.axis_index("s")
        wid = cid * NS + sid

        nch = jnp.where(sid < 15, 25, 15)
        r0 = sid * 3200

        def wait_gather(p):
            pltpu.make_async_copy(
                featPs[0].at[srcc_v.at[0]], rows[p], gsems[p]).wait()

        def wait_scatter(p):
            pltpu.make_async_copy(
                rows[p], acc.at[dstc_v.at[0]], ssems[p]).wait()

        def scale(p, row):
            def scale_body(kk, _):
                av = plsc.load_gather(
                    eeb_v, [jnp.full((16,), row, jnp.int32),
                            jnp.full((16,), kk, jnp.int32)])
                rows[p][kk, pl.ds(0, 16)] = rows[p][kk, pl.ds(0, 16)] * av
                rows[p][kk, pl.ds(16, 16)] = rows[p][kk, pl.ds(16, 16)] * av
                return ()
            lax.fori_loop(0, K, scale_body, (), unroll=8)

        for h in range(H):
            def zb_body(kk, _):
                rows0[kk, pl.ds(0, 16)] = jnp.zeros((16,), jnp.float32)
                rows0[kk, pl.ds(16, 16)] = jnp.zeros((16,), jnp.float32)
                return ()
            lax.fori_loop(0, K, zb_body, (), unroll=4)

            def zero_body(z, _):
                pltpu.sync_copy(rows0, acc.at[pl.ds(r0 + z * K, K)])
                return ()
            lax.fori_loop(0, nch, zero_body, ())

            @pl.when(sid == 15)
            def _():
                pltpu.sync_copy(rows0.at[pl.ds(0, 80)],
                                acc.at[pl.ds(49920, 80)])
            plsc.subcore_barrier()

            def sup_body(s, _):
                row0 = wid * ROWS_PER_TILE + s * SUPROWS
                pltpu.sync_copy(srcR.at[pl.ds(row0, SUPROWS)], srcc_v)
                pltpu.sync_copy(dstR.at[pl.ds(row0, SUPROWS)], dstc_v)
                pltpu.sync_copy(eeP.at[h, pl.ds(row0, SUPROWS)], eeb_v)

                pltpu.async_copy(
                    featPs[h].at[srcc_v.at[0]], rows[0], gsems[0])

                def pair_body(t, _):
                    c0 = 2 * t
                    @pl.when(t > 0)
                    def _():
                        wait_scatter(1)
                    pltpu.async_copy(
                        featPs[h].at[srcc_v.at[c0 + 1]], rows[1], gsems[1])
                    wait_gather(0)
                    scale(0, c0)
                    pltpu.async_copy(
                        rows[0], acc.at[dstc_v.at[c0]], ssems[0], add=True)
                    wait_gather(1)
                    scale(1, c0 + 1)
                    pltpu.async_copy(
                        rows[1], acc.at[dstc_v.at[c0 + 1]], ssems[1],
                        add=True)

                    @pl.when(t < SUPROWS // 2 - 1)
                    def _():
                        wait_scatter(0)
                        pltpu.async_copy(
                            featPs[h].at[srcc_v.at[c0 + 2]], rows[0],
                            gsems[0])
                    return ()
                lax.fori_loop(0, SUPROWS // 2, pair_body, ())
                wait_scatter(0)
                wait_scatter(1)
                return ()
            lax.fori_loop(0, SUPS, sup_body, ())
            plsc.subcore_barrier()

            def copy_body(z, _):
                pltpu.sync_copy(
                    acc.at[pl.ds(r0 + z * K, K)],
                    out.at[cid, h, pl.ds(r0 + z * K, K)])
                return ()
            lax.fori_loop(0, nch, copy_body, ())

            @pl.when(sid == 15)
            def _():
                pltpu.sync_copy(acc.at[pl.ds(49920, 80)],
                                out.at[cid, h, pl.ds(49920, 80)])
            plsc.subcore_barrier()

    scratch = [
        pltpu.VMEM((SUPROWS, K), jnp.int32),
        pltpu.VMEM((SUPROWS, K), jnp.int32),
        pltpu.VMEM((SUPROWS, K), jnp.float32),
        pltpu.VMEM((K, 32), jnp.float32),
        pltpu.VMEM((K, 32), jnp.float32),
        pltpu.VMEM_SHARED((NACC, 32), jnp.float32),
        pltpu.SemaphoreType.DMA,
        pltpu.SemaphoreType.DMA,
        pltpu.SemaphoreType.DMA,
        pltpu.SemaphoreType.DMA,
    ]
    return pl.kernel(
        body,
        out_type=jax.ShapeDtypeStruct((NC, H, N, 32), jnp.float32),
        mesh=_sc_mesh(),
        scratch_types=scratch,
        compiler_params=_SC_PARAMS,
    )


def _emit_tables(feat, al_ref, ar_ref, sel_ref, c_ref, H,
                 featP_refs, elT_ref, erT_ref):
    elT_ref[...] = jnp.dot(feat, al_ref[...],
                           preferred_element_type=jnp.float32)
    erT_ref[...] = jnp.dot(feat, ar_ref[...],
                           preferred_element_type=jnp.float32)
    for h in range(H):
        featP_refs[h][...] = jnp.dot(
            feat, sel_ref[h], preferred_element_type=jnp.float32) + c_ref[...]


def _tc_prep1(x, W, AL, AR, S, c):
    B = 2000
    Fin, Fout = W.shape
    Hh = AL.shape[1]

    def body(x_ref, w_ref, al_ref, ar_ref, sel_ref, c_ref, *out_refs):
        feat = jnp.dot(x_ref[...], w_ref[...],
                       preferred_element_type=jnp.float32)
        _emit_tables(feat, al_ref, ar_ref, sel_ref, c_ref, Hh,
                     out_refs[:Hh], out_refs[Hh], out_refs[Hh + 1])

    return pl.pallas_call(
        body,
        grid=(N // B,),
        in_specs=[
            pl.BlockSpec((B, Fin), lambda i: (i, 0)),
            pl.BlockSpec((Fin, Fout), lambda i: (0, 0)),
            pl.BlockSpec((Fout, Hh), lambda i: (0, 0)),
            pl.BlockSpec((Fout, Hh), lambda i: (0, 0)),
            pl.BlockSpec((Hh, Fout, 32), lambda i: (0, 0, 0)),
            pl.BlockSpec((1, 32), lambda i: (0, 0)),
        ],
        out_specs=(
            [pl.BlockSpec((B, 32), lambda i: (i, 0)) for _ in range(Hh)]
            + [pl.BlockSpec((B, Hh), lambda i: (i, 0)),
               pl.BlockSpec((B, Hh), lambda i: (i, 0))]
        ),
        out_shape=(
            [jax.ShapeDtypeStruct((N, 32), jnp.float32) for _ in range(Hh)]
            + [jax.ShapeDtypeStruct((N, Hh), jnp.float32),
               jax.ShapeDtypeStruct((N, Hh), jnp.float32)]
        ),
    )(x, W, AL, AR, S, c)


def _tc_prep_next(p0, p1, b, W, AL, AR, S, c):
    B = 2000
    Hp = p0.shape[0]
    Fin, Fout = W.shape
    Hh = AL.shape[1]

    def body(p0_ref, p1_ref, b_ref, w_ref, al_ref, ar_ref, sel_ref, c_ref,
             *out_refs):
        num = p0_ref[...] + p1_ref[...]
        den = num[:, :, 25:26] + 1e-16
        nrm = num / den
        hcat = jnp.concatenate([nrm[h, :, :25] for h in range(Hp)],
                               axis=-1) + b_ref[...]
        feat = jnp.dot(hcat, w_ref[...],
                       preferred_element_type=jnp.float32)
        _emit_tables(feat, al_ref, ar_ref, sel_ref, c_ref, Hh,
                     out_refs[:Hh], out_refs[Hh], out_refs[Hh + 1])

    return pl.pallas_call(
        body,
        grid=(N // B,),
        in_specs=[
            pl.BlockSpec((Hp, B, 32), lambda i: (0, i, 0)),
            pl.BlockSpec((Hp, B, 32), lambda i: (0, i, 0)),
            pl.BlockSpec((1, Fin), lambda i: (0, 0)),
            pl.BlockSpec((Fin, Fout), lambda i: (0, 0)),
            pl.BlockSpec((Fout, Hh), lambda i: (0, 0)),
            pl.BlockSpec((Fout, Hh), lambda i: (0, 0)),
            pl.BlockSpec((Hh, Fout, 32), lambda i: (0, 0, 0)),
            pl.BlockSpec((1, 32), lambda i: (0, 0)),
        ],
        out_specs=(
            [pl.BlockSpec((B, 32), lambda i: (i, 0)) for _ in range(Hh)]
            + [pl.BlockSpec((B, Hh), lambda i: (i, 0)),
               pl.BlockSpec((B, Hh), lambda i: (i, 0))]
        ),
        out_shape=(
            [jax.ShapeDtypeStruct((N, 32), jnp.float32) for _ in range(Hh)]
            + [jax.ShapeDtypeStruct((N, Hh), jnp.float32),
               jax.ShapeDtypeStruct((N, Hh), jnp.float32)]
        ),
    )(p0, p1, b, W, AL, AR, S, c)


def _tc_final(p0, p1, b, Wf, bf):
    B = 2000

    def body(p0_ref, p1_ref, b_ref, wf_ref, bf_ref, out_ref):
        num = p0_ref[...] + p1_ref[...]
        den = num[:, :, 25:26] + 1e-16
        h3 = (num / den)[0, :, :25] + b_ref[...]
        z = jnp.dot(h3, wf_ref[...],
                    preferred_element_type=jnp.float32) + bf_ref[...]
        out_ref[...] = 1.0 / (1.0 + jnp.exp(-z))

    return pl.pallas_call(
        body,
        grid=(N // B,),
        in_specs=[
            pl.BlockSpec((1, B, 32), lambda i: (0, i, 0)),
            pl.BlockSpec((1, B, 32), lambda i: (0, i, 0)),
            pl.BlockSpec((1, 25), lambda i: (0, 0)),
            pl.BlockSpec((25, 1), lambda i: (0, 0)),
            pl.BlockSpec((1, 1), lambda i: (0, 0)),
        ],
        out_specs=pl.BlockSpec((B, 1), lambda i: (i, 0)),
        out_shape=jax.ShapeDtypeStruct((N, 1), jnp.float32),
    )(p0, p1, b, Wf, bf)


def _attn_mats(al, ar):
    H, D = al.shape
    AL = jnp.zeros((H * D, H), jnp.float32)
    AR = jnp.zeros((H * D, H), jnp.float32)
    S = jnp.zeros((H, H * D, 32), jnp.float32)
    for h in range(H):
        AL = AL.at[h * D:(h + 1) * D, h].set(al[h])
        AR = AR.at[h * D:(h + 1) * D, h].set(ar[h])
        S = S.at[h, h * D:(h + 1) * D, :D].set(jnp.eye(D, dtype=jnp.float32))
    c = jnp.zeros((1, 32), jnp.float32).at[0, 25].set(1.0)
    return AL, AR, S, c


def kernel(features, graph, W1, al1, ar1, b1, W2, al2, ar2, b2,
           W3, al3, ar3, b3, Wf, bf):
    pad = EPAD - E
    src_p = jnp.pad(graph[0], (0, pad),
                    mode="wrap").reshape(EPAD // K, K)
    dst_p = (jnp.pad(graph[1], (0, pad), constant_values=0)
             .at[E:].set(N + (jnp.arange(pad, dtype=jnp.int32) % NJUNK))
             .reshape(EPAD // K, K))

    AL1, AR1, S1, c1 = _attn_mats(al1, ar1)
    AL2, AR2, S2, c2 = _attn_mats(al2, ar2)
    AL3, AR3, S3, c3 = _attn_mats(al3, ar3)

    *featPs, el, er = _tc_prep1(features, W1, AL1, AR1, S1, c1)
    ee = _sc_ee_kernel(4)(src_p, dst_p, el.T, er.T)
    p = _sc_agg_kernel(4)(src_p, dst_p, ee, *featPs)

    *featPs, el, er = _tc_prep_next(
        p[0], p[1], b1.reshape(1, -1), W2, AL2, AR2, S2, c2)
    ee = _sc_ee_kernel(2)(src_p, dst_p, el.T, er.T)
    p = _sc_agg_kernel(2)(src_p, dst_p, ee, *featPs)

    *featPs, el, er = _tc_prep_next(
        p[0], p[1], b2.reshape(1, -1), W3, AL3, AR3, S3, c3)
    ee = _sc_ee_kernel(1)(src_p, dst_p, el.T, er.T)
    p = _sc_agg_kernel(1)(src_p, dst_p, ee, *featPs)

    return _tc_final(p[0], p[1], b3.reshape(1, -1), Wf, bf.reshape(1, 1))


# speedup vs baseline: 74.8212x; 1.0426x over previous
import functools

import jax
import jax.numpy as jnp
from jax import lax
from jax.experimental import pallas as pl
from jax.experimental.pallas import tpu as pltpu
from jax.experimental.pallas import tpu_sc as plsc

N = 50000
E = 800000
NC = 2
NS = 16
NW = NC * NS
K = 128
SUPROWS = 40
EPT = 25600
EPAD = NW * EPT
EPADK = EPAD // K
ROWS_PER_TILE = EPT // K
SUPS = ROWS_PER_TILE // SUPROWS
NJUNK = 512
NTAB = N + NJUNK
NACC = N + NJUNK


def _sc_mesh():
    return plsc.VectorSubcoreMesh(
        core_axis_name="c", subcore_axis_name="s", num_cores=NC,
        num_subcores=NS)


_SC_PARAMS = pltpu.CompilerParams(
    needs_layout_passes=False, use_tc_tiling_on_sc=False)


def _sc_ee_kernel(H):
    def body(*refs):
        srcR, dstR, elT, erT = refs[:4]
        out = refs[4]
        el_v, er_v, srcc_v, dstc_v, eeb_v = refs[5:]

        cid = lax.axis_index("c")
        sid = lax.axis_index("s")
        wid = cid * NS + sid

        def zt_body(kk, _):
            el_v[pl.ds(N + kk * 16, 16)] = jnp.zeros((16,), jnp.float32)
            er_v[pl.ds(N + kk * 16, 16)] = jnp.zeros((16,), jnp.float32)
            return ()
        lax.fori_loop(0, NJUNK // 16, zt_body, (), unroll=4)

        for h in range(H):
            pltpu.sync_copy(elT.at[h], el_v.at[pl.ds(0, N)])
            pltpu.sync_copy(erT.at[h], er_v.at[pl.ds(0, N)])

            def sup_body(s, _):
                row0 = wid * ROWS_PER_TILE + s * SUPROWS
                pltpu.sync_copy(srcR.at[pl.ds(row0, SUPROWS)], srcc_v)
                pltpu.sync_copy(dstR.at[pl.ds(row0, SUPROWS)], dstc_v)

                def chunk_body(j, _):
                    for g in range(8):
                        s16 = srcc_v[j, pl.ds(g * 16, 16)]
                        d16 = dstc_v[j, pl.ds(g * 16, 16)]
                        el16 = plsc.load_gather(el_v, [s16])
                        er16 = plsc.load_gather(er_v, [d16])
                        t = el16 + er16
                        e = jnp.maximum(t, 0.2 * t)
                        eeb_v[j, pl.ds(g * 16, 16)] = jnp.exp(e)
                    return ()
                lax.fori_loop(0, SUPROWS, chunk_body, ())
                pltpu.sync_copy(eeb_v, out.at[h, pl.ds(row0, SUPROWS)])
                return ()
            lax.fori_loop(0, SUPS, sup_body, ())

    scratch = [
        pltpu.VMEM((NTAB,), jnp.float32),
        pltpu.VMEM((NTAB,), jnp.float32),
        pltpu.VMEM((SUPROWS, K), jnp.int32),
        pltpu.VMEM((SUPROWS, K), jnp.int32),
        pltpu.VMEM((SUPROWS, K), jnp.float32),
    ]
    return pl.kernel(
        body,
        out_type=jax.ShapeDtypeStruct((H, EPADK, K), jnp.float32),
        mesh=_sc_mesh(),
        scratch_types=scratch,
        compiler_params=_SC_PARAMS,
    )


def _sc_agg_kernel(H):
    NB = 4
    BSUP = 8
    BSUPS = ROWS_PER_TILE // BSUP

    def body(*refs):
        srcR, dstR, eeP = refs[0], refs[1], refs[2]
        featPs = refs[3:3 + H]
        out0, out1 = refs[3 + H], refs[4 + H]
        srcc_v, dstc_v, eeb_v = refs[5 + H:8 + H]
        rows = refs[8 + H:8 + H + NB]
        acc = refs[8 + H + NB]
        gsems = refs[9 + H + NB:9 + H + 2 * NB]
        ssems = refs[9 + H + 2 * NB:9 + H + 3 * NB]

        cid = lax.axis_index("c")
        sid = lax.axis_index("s")
        wid = cid * NS + sid

        nch = jnp.where(sid < 15, 25, 15)
        r0 = sid * 3200

        def wait_scatter(p):
            pltpu.make_async_copy(
                rows[p], acc.at[dstc_v.at[0]], ssems[p]).wait()

        def scale(p, row):
            def scale_body(kk, _):
                av = plsc.load_gather(
                    eeb_v, [jnp.full((16,), row, jnp.int32),
                            jnp.full((16,), kk, jnp.int32)])
                rows[p][kk, pl.ds(0, 16)] = rows[p][kk, pl.ds(0, 16)] * av
                rows[p][kk, pl.ds(16, 16)] = rows[p][kk, pl.ds(16, 16)] * av
                return ()
            lax.fori_loop(0, K, scale_body, (), unroll=8)

        for h in range(H):
            def zb_body(kk, _):
                rows[0][kk, pl.ds(0, 16)] = jnp.zeros((16,), jnp.float32)
                rows[0][kk, pl.ds(16, 16)] = jnp.zeros((16,), jnp.float32)
                return ()
            lax.fori_loop(0, K, zb_body, (), unroll=4)

            def zero_body(z, _):
                pltpu.sync_copy(rows[0], acc.at[pl.ds(r0 + z * K, K)])
                return ()
            lax.fori_loop(0, nch, zero_body, ())

            @pl.when(sid == 15)
            def _():
                pltpu.sync_copy(rows[0].at[pl.ds(0, 80)],
                                acc.at[pl.ds(49920, 80)])
            plsc.subcore_barrier()

            def sup_body(s, _):
                @pl.when(s > 0)
                def _():
                    for p in range(NB):
                        wait_scatter(p)

                row0 = wid * ROWS_PER_TILE + s * BSUP
                pltpu.sync_copy(srcR.at[pl.ds(row0, BSUP)], srcc_v)
                pltpu.sync_copy(dstR.at[pl.ds(row0, BSUP)], dstc_v)
                pltpu.sync_copy(eeP.at[h, pl.ds(row0, BSUP)], eeb_v)

                gd = [None] * BSUP
                sd = [None] * BSUP
                for c in range(NB):
                    gd[c] = pltpu.async_copy(
                        featPs[h].at[srcc_v.at[c]], rows[c], gsems[c])
                for c in range(BSUP):
                    if c >= 1 and c + NB - 1 < BSUP:
                        sd[c - 1].wait()
                        gd[c + NB - 1] = pltpu.async_copy(
                            featPs[h].at[srcc_v.at[c + NB - 1]],
                            rows[(c + NB - 1) % NB], gsems[(c + NB - 1) % NB])
                    gd[c].wait()
                    scale(c % NB, c)
                    sd[c] = pltpu.async_copy(
                        rows[c % NB], acc.at[dstc_v.at[c]], ssems[c % NB],
                        add=True)
                return ()
            lax.fori_loop(0, BSUPS, sup_body, ())
            for p in range(NB):
                wait_scatter(p)
            plsc.subcore_barrier()

            def copy_to(dst):
                def copy_body(z, _):
                    pltpu.sync_copy(
                        acc.at[pl.ds(r0 + z * K, K)],
                        dst.at[h, pl.ds(r0 + z * K, K)])
                    return ()
                lax.fori_loop(0, nch, copy_body, ())

                @pl.when(sid == 15)
                def _():
                    pltpu.sync_copy(acc.at[pl.ds(49920, 80)],
                                    dst.at[h, pl.ds(49920, 80)])

            @pl.when(cid == 0)
            def _():
                copy_to(out0)

            @pl.when(cid == 1)
            def _():
                copy_to(out1)
            plsc.subcore_barrier()

    scratch = (
        [pltpu.VMEM((BSUP, K), jnp.int32),
         pltpu.VMEM((BSUP, K), jnp.int32),
         pltpu.VMEM((BSUP, K), jnp.float32)]
        + [pltpu.VMEM((K, 32), jnp.float32) for _ in range(NB)]
        + [pltpu.VMEM_SHARED((NACC, 32), jnp.float32)]
        + [pltpu.SemaphoreType.DMA] * (2 * NB)
    )
    return pl.kernel(
        body,
        out_type=[jax.ShapeDtypeStruct((H, N, 32), jnp.float32),
                  jax.ShapeDtypeStruct((H, N, 32), jnp.float32)],
        mesh=_sc_mesh(),
        scratch_types=scratch,
        compiler_params=_SC_PARAMS,
    )


def _emit_tables(feat, al_ref, ar_ref, sel_ref, c_ref, H,
                 featP_refs, elT_ref, erT_ref):
    elT_ref[...] = jnp.dot(feat, al_ref[...],
                           preferred_element_type=jnp.float32)
    erT_ref[...] = jnp.dot(feat, ar_ref[...],
                           preferred_element_type=jnp.float32)
    for h in range(H):
        featP_refs[h][...] = jnp.dot(
            feat, sel_ref[h], preferred_element_type=jnp.float32) + c_ref[...]


def _tc_prep1(x, W, AL, AR, S, c):
    B = 2000
    Fin, Fout = W.shape
    Hh = AL.shape[1]

    def body(x_ref, w_ref, al_ref, ar_ref, sel_ref, c_ref, *out_refs):
        feat = jnp.dot(x_ref[...], w_ref[...],
                       preferred_element_type=jnp.float32)
        _emit_tables(feat, al_ref, ar_ref, sel_ref, c_ref, Hh,
                     out_refs[:Hh], out_refs[Hh], out_refs[Hh + 1])

    return pl.pallas_call(
        body,
        grid=(N // B,),
        in_specs=[
            pl.BlockSpec((B, Fin), lambda i: (i, 0)),
            pl.BlockSpec((Fin, Fout), lambda i: (0, 0)),
            pl.BlockSpec((Fout, Hh), lambda i: (0, 0)),
            pl.BlockSpec((Fout, Hh), lambda i: (0, 0)),
            pl.BlockSpec((Hh, Fout, 32), lambda i: (0, 0, 0)),
            pl.BlockSpec((1, 32), lambda i: (0, 0)),
        ],
        out_specs=(
            [pl.BlockSpec((B, 32), lambda i: (i, 0)) for _ in range(Hh)]
            + [pl.BlockSpec((B, Hh), lambda i: (i, 0)),
               pl.BlockSpec((B, Hh), lambda i: (i, 0))]
        ),
        out_shape=(
            [jax.ShapeDtypeStruct((N, 32), jnp.float32) for _ in range(Hh)]
            + [jax.ShapeDtypeStruct((N, Hh), jnp.float32),
               jax.ShapeDtypeStruct((N, Hh), jnp.float32)]
        ),
    )(x, W, AL, AR, S, c)


def _tc_prep_next(p0, p1, b, W, AL, AR, S, c):
    B = 2000
    Hp = p0.shape[0]
    Fin, Fout = W.shape
    Hh = AL.shape[1]

    def body(p0_ref, p1_ref, b_ref, w_ref, al_ref, ar_ref, sel_ref, c_ref,
             *out_refs):
        num = p0_ref[...] + p1_ref[...]
        den = num[:, :, 25:26] + 1e-16
        nrm = num / den
        hcat = jnp.concatenate([nrm[h, :, :25] for h in range(Hp)],
                               axis=-1) + b_ref[...]
        feat = jnp.dot(hcat, w_ref[...],
                       preferred_element_type=jnp.float32)
        _emit_tables(feat, al_ref, ar_ref, sel_ref, c_ref, Hh,
                     out_refs[:Hh], out_refs[Hh], out_refs[Hh + 1])

    return pl.pallas_call(
        body,
        grid=(N // B,),
        in_specs=[
            pl.BlockSpec((Hp, B, 32), lambda i: (0, i, 0)),
            pl.BlockSpec((Hp, B, 32), lambda i: (0, i, 0)),
            pl.BlockSpec((1, Fin), lambda i: (0, 0)),
            pl.BlockSpec((Fin, Fout), lambda i: (0, 0)),
            pl.BlockSpec((Fout, Hh), lambda i: (0, 0)),
            pl.BlockSpec((Fout, Hh), lambda i: (0, 0)),
            pl.BlockSpec((Hh, Fout, 32), lambda i: (0, 0, 0)),
            pl.BlockSpec((1, 32), lambda i: (0, 0)),
        ],
        out_specs=(
            [pl.BlockSpec((B, 32), lambda i: (i, 0)) for _ in range(Hh)]
            + [pl.BlockSpec((B, Hh), lambda i: (i, 0)),
               pl.BlockSpec((B, Hh), lambda i: (i, 0))]
        ),
        out_shape=(
            [jax.ShapeDtypeStruct((N, 32), jnp.float32) for _ in range(Hh)]
            + [jax.ShapeDtypeStruct((N, Hh), jnp.float32),
               jax.ShapeDtypeStruct((N, Hh), jnp.float32)]
        ),
    )(p0, p1, b, W, AL, AR, S, c)


def _tc_final(p0, p1, b, Wf, bf):
    B = 2000

    def body(p0_ref, p1_ref, b_ref, wf_ref, bf_ref, out_ref):
        num = p0_ref[...] + p1_ref[...]
        den = num[:, :, 25:26] + 1e-16
        h3 = (num / den)[0, :, :25] + b_ref[...]
        z = jnp.dot(h3, wf_ref[...],
                    preferred_element_type=jnp.float32) + bf_ref[...]
        out_ref[...] = 1.0 / (1.0 + jnp.exp(-z))

    return pl.pallas_call(
        body,
        grid=(N // B,),
        in_specs=[
            pl.BlockSpec((1, B, 32), lambda i: (0, i, 0)),
            pl.BlockSpec((1, B, 32), lambda i: (0, i, 0)),
            pl.BlockSpec((1, 25), lambda i: (0, 0)),
            pl.BlockSpec((25, 1), lambda i: (0, 0)),
            pl.BlockSpec((1, 1), lambda i: (0, 0)),
        ],
        out_specs=pl.BlockSpec((B, 1), lambda i: (i, 0)),
        out_shape=jax.ShapeDtypeStruct((N, 1), jnp.float32),
    )(p0, p1, b, Wf, bf)


def _attn_mats(al, ar):
    H, D = al.shape
    AL = jnp.zeros((H * D, H), jnp.float32)
    AR = jnp.zeros((H * D, H), jnp.float32)
    S = jnp.zeros((H, H * D, 32), jnp.float32)
    for h in range(H):
        AL = AL.at[h * D:(h + 1) * D, h].set(al[h])
        AR = AR.at[h * D:(h + 1) * D, h].set(ar[h])
        S = S.at[h, h * D:(h + 1) * D, :D].set(jnp.eye(D, dtype=jnp.float32))
    c = jnp.zeros((1, 32), jnp.float32).at[0, 25].set(1.0)
    return AL, AR, S, c


def kernel(features, graph, W1, al1, ar1, b1, W2, al2, ar2, b2,
           W3, al3, ar3, b3, Wf, bf):
    pad = EPAD - E
    src_p = jnp.pad(graph[0], (0, pad),
                    mode="wrap").reshape(EPAD // K, K)
    dst_p = (jnp.pad(graph[1], (0, pad), constant_values=0)
             .at[E:].set(N + (jnp.arange(pad, dtype=jnp.int32) % NJUNK))
             .reshape(EPAD // K, K))

    AL1, AR1, S1, c1 = _attn_mats(al1, ar1)
    AL2, AR2, S2, c2 = _attn_mats(al2, ar2)
    AL3, AR3, S3, c3 = _attn_mats(al3, ar3)

    *featPs, el, er = _tc_prep1(features, W1, AL1, AR1, S1, c1)
    ee = _sc_ee_kernel(4)(src_p, dst_p, el.T, er.T)
    p = _sc_agg_kernel(4)(src_p, dst_p, ee, *featPs)

    *featPs, el, er = _tc_prep_next(
        p[0], p[1], b1.reshape(1, -1), W2, AL2, AR2, S2, c2)
    ee = _sc_ee_kernel(2)(src_p, dst_p, el.T, er.T)
    p = _sc_agg_kernel(2)(src_p, dst_p, ee, *featPs)

    *featPs, el, er = _tc_prep_next(
        p[0], p[1], b2.reshape(1, -1), W3, AL3, AR3, S3, c3)
    ee = _sc_ee_kernel(1)(src_p, dst_p, el.T, er.T)
    p = _sc_agg_kernel(1)(src_p, dst_p, ee, *featPs)

    return _tc_final(p[0], p[1], b3.reshape(1, -1), Wf, bf.reshape(1, 1))
